# baseline (speedup 1.0000x reference)
import functools

import jax
import jax.numpy as jnp
from jax.experimental import pallas as pl
from jax.experimental.pallas import tpu as pltpu

D = 768
E = 64
R = 8
ER = E * R
K = 2
SCALING = 16.0 / 8.0

TB = 1024


def _gate_from(xb, wrt, br):
    logits = jnp.dot(xb, wrt, preferred_element_type=jnp.float32) + br
    m1 = jnp.max(logits, axis=1, keepdims=True)
    mask1 = logits == m1
    l2 = jnp.where(mask1, jnp.float32(-3.0e38), logits)
    m2 = jnp.max(l2, axis=1, keepdims=True)
    mask2 = l2 == m2
    e21 = jnp.exp(m2 - m1)
    w1 = 1.0 / (1.0 + e21)
    w2 = e21 * w1
    return jnp.where(mask1, w1, 0.0) + jnp.where(mask2, w2, 0.0)


def _fused_body(x_ref, xn_ref, waf_ref, bb_ref, wrt_ref, br_ref, bf_ref,
                o_ref, g_ref):
    i = pl.program_id(0)

    @pl.when(i == 0)
    def _prologue():
        g_ref[...] = _gate_from(x_ref[...], wrt_ref[...], br_ref[...])

    xb16 = x_ref[...].astype(jnp.bfloat16)
    z = jnp.dot(xb16, waf_ref[...], preferred_element_type=jnp.float32)
    base = z[:, :D] + bb_ref[...]
    h = z[:, D:]
    gate512 = jnp.concatenate([g_ref[...]] * R, axis=1)
    hg = (h * gate512).astype(jnp.bfloat16)
    lora = jnp.dot(hg, bf_ref[...], preferred_element_type=jnp.float32)
    o_ref[...] = base + SCALING * lora

    g_ref[...] = _gate_from(xn_ref[...], wrt_ref[...], br_ref[...])


@functools.partial(jax.jit, static_argnames=("interpret",))
def kernel(x, W_base, b_base, W_router, b_router, A, B, interpret=False):
    T = x.shape[0]
    nb = T // TB
    wrt = W_router.T
    waf = jnp.concatenate(
        [W_base.T, A.transpose(2, 1, 0).reshape(D, ER)],
        axis=1).astype(jnp.bfloat16)
    bf = B.transpose(2, 0, 1).reshape(ER, D).astype(jnp.bfloat16)
    bb = b_base.reshape(1, D)
    br = b_router.reshape(1, E)

    out = pl.pallas_call(
        _fused_body,
        grid=(nb,),
        in_specs=[
            pl.BlockSpec((TB, D), lambda i: (i, 0)),
            pl.BlockSpec((TB, D), lambda i: (jnp.minimum(i + 1, nb - 1), 0)),
            pl.BlockSpec((D, D + ER), lambda i: (0, 0)),
            pl.BlockSpec((1, D), lambda i: (0, 0)),
            pl.BlockSpec((D, E), lambda i: (0, 0)),
            pl.BlockSpec((1, E), lambda i: (0, 0)),
            pl.BlockSpec((ER, D), lambda i: (0, 0)),
        ],
        out_specs=pl.BlockSpec((TB, D), lambda i: (i, 0)),
        out_shape=jax.ShapeDtypeStruct((T, D), jnp.float32),
        scratch_shapes=[pltpu.VMEM((TB, E), jnp.float32)],
        interpret=interpret,
    )(x, x, waf, bb, wrt, br, bf)
    return out


# speedup vs baseline: 1.2874x; 1.2874x over previous
import functools

import jax
import jax.numpy as jnp
from jax.experimental import pallas as pl

D = 768
E = 64
R = 8
ER = E * R
K = 2
SCALING = 16.0 / 8.0

TB = 2048


def _fused_body(x_ref, waf_ref, bb_ref, wrt_ref, br_ref, bf_ref, o_ref):
    xb = x_ref[...]

    logits = jnp.dot(xb, wrt_ref[...], preferred_element_type=jnp.float32)
    logits = logits + br_ref[...]

    m1 = jnp.max(logits, axis=1, keepdims=True)
    mask1 = logits == m1
    l2 = jnp.where(mask1, jnp.float32(-3.0e38), logits)
    m2 = jnp.max(l2, axis=1, keepdims=True)
    mask2 = l2 == m2
    e21 = jnp.exp(m2 - m1)
    w1 = 1.0 / (1.0 + e21)
    w2 = e21 * w1
    gate = jnp.where(mask1, w1, 0.0) + jnp.where(mask2, w2, 0.0)

    xb16 = xb.astype(jnp.bfloat16)
    z = jnp.dot(xb16, waf_ref[...], preferred_element_type=jnp.float32)
    base = z[:, :D] + bb_ref[...]
    h = z[:, D:]
    gate512 = jnp.concatenate([gate] * R, axis=1)
    hg = (h * gate512).astype(jnp.bfloat16)
    lora = jnp.dot(hg, bf_ref[...], preferred_element_type=jnp.float32)

    o_ref[...] = base + SCALING * lora


@functools.partial(jax.jit, static_argnames=("interpret",))
def kernel(x, W_base, b_base, W_router, b_router, A, B, interpret=False):
    T = x.shape[0]
    wrt = W_router.T
    waf = jnp.concatenate(
        [W_base.T, A.transpose(2, 1, 0).reshape(D, ER)],
        axis=1).astype(jnp.bfloat16)
    bf = B.transpose(2, 0, 1).reshape(ER, D).astype(jnp.bfloat16)
    bb = b_base.reshape(1, D)
    br = b_router.reshape(1, E)

    grid = (T // TB,)
    out = pl.pallas_call(
        _fused_body,
        grid=grid,
        in_specs=[
            pl.BlockSpec((TB, D), lambda i: (i, 0)),
            pl.BlockSpec((D, D + ER), lambda i: (0, 0)),
            pl.BlockSpec((1, D), lambda i: (0, 0)),
            pl.BlockSpec((D, E), lambda i: (0, 0)),
            pl.BlockSpec((1, E), lambda i: (0, 0)),
            pl.BlockSpec((ER, D), lambda i: (0, 0)),
        ],
        out_specs=pl.BlockSpec((TB, D), lambda i: (i, 0)),
        out_shape=jax.ShapeDtypeStruct((T, D), jnp.float32),
        interpret=interpret,
    )(x, waf, bb, wrt, br, bf)
    return out
